# Initial kernel scaffold; baseline (speedup 1.0000x reference)
import jax
import jax.numpy as jnp
from jax.experimental import pallas as pl


def kernel(pos, mapping, mapping_batch, atom_types, theta_0, theta_1, theta_2, k_0, k_1, k_2):
    raise NotImplementedError("write your pallas kernel here")


# speedup vs baseline: 134.7162x; 134.7162x over previous
import functools
import jax
import jax.numpy as jnp
from jax import lax
from jax.experimental import pallas as pl
from jax.experimental.pallas import tpu as pltpu
from jax.experimental.pallas import tpu_sc as plsc

N_B = 64
N_T = 25
CHUNK = 128
NW = 32
E_REAL = 800000
PER_W = 25088
N_CHUNKS = PER_W // CHUNK
E_PAD = PER_W * NW


def _rsqrt(x):
    i = lax.bitcast_convert_type(x, jnp.int32)
    i = jnp.int32(0x5F3759DF) - lax.shift_right_arithmetic(i, 1)
    y = lax.bitcast_convert_type(i, jnp.float32)
    for _ in range(3):
        y = y * (1.5 - 0.5 * x * y * y)
    return y


def _body(m0_h, m1_h, m2_h, m3_h, mb_h, pos_h, types_h, tab_h, out_h,
          types_v, m0_v, m1_v, m2_v, m3_v, mb_v, tix_v,
          p0_v, p1_v, p2_v, p3_v, tab_v, acc_v, outp_v, sem):
    cid = lax.axis_index("c")
    sid = lax.axis_index("s")
    wid = sid * 2 + cid
    base_w = wid * PER_W

    pltpu.sync_copy(types_h, types_v)

    zero = jnp.zeros((16,), jnp.float32)
    for i in range(64):
        acc_v[pl.ds(i * 16, 16)] = zero

    iota = lax.iota(jnp.int32, 16)

    def chunk_body(ci, carry):
        base = base_w + ci * CHUNK
        cps = [
            pltpu.async_copy(m0_h.at[pl.ds(base, CHUNK)], m0_v, sem),
            pltpu.async_copy(m1_h.at[pl.ds(base, CHUNK)], m1_v, sem),
            pltpu.async_copy(m2_h.at[pl.ds(base, CHUNK)], m2_v, sem),
            pltpu.async_copy(m3_h.at[pl.ds(base, CHUNK)], m3_v, sem),
            pltpu.async_copy(mb_h.at[pl.ds(base, CHUNK)], mb_v, sem),
        ]
        for c in cps:
            c.wait()
        pcs = [
            pltpu.async_copy(pos_h.at[m0_v], p0_v, sem),
            pltpu.async_copy(pos_h.at[m1_v], p1_v, sem),
            pltpu.async_copy(pos_h.at[m2_v], p2_v, sem),
            pltpu.async_copy(pos_h.at[m3_v], p3_v, sem),
        ]
        for j in range(CHUNK // 16):
            sl = pl.ds(j * 16, 16)
            it0 = plsc.load_gather(types_v, [m0_v[sl]])
            it1 = plsc.load_gather(types_v, [m1_v[sl]])
            it2 = plsc.load_gather(types_v, [m2_v[sl]])
            it3 = plsc.load_gather(types_v, [m3_v[sl]])
            tix_v[sl] = ((it0 * N_T + it1) * N_T + it2) * N_T + it3
        tc = pltpu.async_copy(tab_h.at[tix_v], tab_v, sem)
        for c in pcs:
            c.wait()
        tc.wait()

        for j in range(CHUNK // 16):
            sl = pl.ds(j * 16, 16)
            rows = iota + (j * 16)
            c0 = jnp.full((16,), 0, jnp.int32)
            c1i = jnp.full((16,), 1, jnp.int32)
            c2i = jnp.full((16,), 2, jnp.int32)
            x0 = plsc.load_gather(p0_v, [rows, c0])
            y0 = plsc.load_gather(p0_v, [rows, c1i])
            z0 = plsc.load_gather(p0_v, [rows, c2i])
            x1 = plsc.load_gather(p1_v, [rows, c0])
            y1 = plsc.load_gather(p1_v, [rows, c1i])
            z1 = plsc.load_gather(p1_v, [rows, c2i])
            x2 = plsc.load_gather(p2_v, [rows, c0])
            y2 = plsc.load_gather(p2_v, [rows, c1i])
            z2 = plsc.load_gather(p2_v, [rows, c2i])
            x3 = plsc.load_gather(p3_v, [rows, c0])
            y3 = plsc.load_gather(p3_v, [rows, c1i])
            z3 = plsc.load_gather(p3_v, [rows, c2i])
            b0x, b0y, b0z = x0 - x1, y0 - y1, z0 - z1
            b1x, b1y, b1z = x2 - x1, y2 - y1, z2 - z1
            b2x, b2y, b2z = x3 - x2, y3 - y2, z3 - z2
            nsq = b1x * b1x + b1y * b1y + b1z * b1z
            nz = nsq > 0.0
            inv = jnp.where(nz, _rsqrt(jnp.where(nz, nsq, 1.0)), 0.0)
            ux, uy, uz = b1x * inv, b1y * inv, b1z * inv
            d0 = b0x * ux + b0y * uy + b0z * uz
            d2 = b2x * ux + b2y * uy + b2z * uz
            vx, vy, vz = b0x - d0 * ux, b0y - d0 * uy, b0z - d0 * uz
            wx, wy, wz = b2x - d2 * ux, b2y - d2 * uy, b2z - d2 * uz
            x = vx * wx + vy * wy + vz * wz
            cxx = uy * vz - uz * vy
            cxy = uz * vx - ux * vz
            cxz = ux * vy - uy * vx
            y = cxx * wx + cxy * wy + cxz * wz
            rsq = x * x + y * y
            rnz = rsq > 0.0
            irr = jnp.where(rnz, _rsqrt(jnp.where(rnz, rsq, 1.0)), 0.0)
            c1 = jnp.where(rnz, x * irr, 1.0)
            s1 = y * irr
            c2 = c1 * c1 - s1 * s1
            s2 = 2.0 * c1 * s1
            c3 = c1 * c2 - s1 * s2
            s3 = s1 * c2 + c1 * s2
            K = plsc.load_gather(tab_v, [rows, c0])
            A1 = plsc.load_gather(tab_v, [rows, c1i])
            B1 = plsc.load_gather(tab_v, [rows, c2i])
            A2 = plsc.load_gather(tab_v, [rows, jnp.full((16,), 3, jnp.int32)])
            B2 = plsc.load_gather(tab_v, [rows, jnp.full((16,), 4, jnp.int32)])
            A3 = plsc.load_gather(tab_v, [rows, jnp.full((16,), 5, jnp.int32)])
            B3 = plsc.load_gather(tab_v, [rows, jnp.full((16,), 6, jnp.int32)])
            V = K - (A1 * c1 + B1 * s1 + A2 * c2 + B2 * s2 + A3 * c3 + B3 * s3)
            gidx = base + (j * 16) + iota
            V = jnp.where(gidx < E_REAL, V, 0.0)
            plsc.addupdate_scatter(acc_v, [iota * N_B + mb_v[sl]], V)
        return carry

    lax.fori_loop(0, N_CHUNKS, chunk_body, 0)

    for p in range(4):
        s = jnp.zeros((16,), jnp.float32)
        for r in range(16):
            s = s + acc_v[pl.ds(r * N_B + p * 16, 16)]
        outp_v[pl.ds(p * 16, 16)] = s
    pltpu.sync_copy(outp_v, out_h.at[pl.ds(wid * N_B, N_B)])


@jax.jit
def kernel(pos, mapping, mapping_batch, atom_types,
           theta_0, theta_1, theta_2, k_0, k_1, k_2):
    n_nodes = pos.shape[0]
    flat = (N_T ** 4,)
    K = (k_0 + k_1 + k_2).reshape(flat)
    cols = [K]
    for t, k in ((theta_0, k_0), (theta_1, k_1), (theta_2, k_2)):
        cols.append((k * jnp.cos(t)).reshape(flat))
        cols.append((k * jnp.sin(t)).reshape(flat))
    cols.append(jnp.zeros(flat, jnp.float32))
    tab = jnp.stack(cols, axis=-1)
    pos4 = jnp.concatenate(
        [pos, jnp.zeros((n_nodes, 1), jnp.float32)], axis=1)
    pad = E_PAD - E_REAL
    m = jnp.pad(mapping, ((0, 0), (0, pad)))
    mb = jnp.pad(mapping_batch, (0, pad))

    grid_kernel = pl.kernel(
        _body,
        out_type=jax.ShapeDtypeStruct((NW * N_B,), jnp.float32),
        mesh=plsc.VectorSubcoreMesh(core_axis_name="c", subcore_axis_name="s"),
        compiler_params=pltpu.CompilerParams(
            use_tc_tiling_on_sc=False, needs_layout_passes=False),
        scratch_types=[
            pltpu.VMEM((n_nodes,), jnp.int32),
            pltpu.VMEM((CHUNK,), jnp.int32),
            pltpu.VMEM((CHUNK,), jnp.int32),
            pltpu.VMEM((CHUNK,), jnp.int32),
            pltpu.VMEM((CHUNK,), jnp.int32),
            pltpu.VMEM((CHUNK,), jnp.int32),
            pltpu.VMEM((CHUNK,), jnp.int32),
            pltpu.VMEM((CHUNK, 4), jnp.float32),
            pltpu.VMEM((CHUNK, 4), jnp.float32),
            pltpu.VMEM((CHUNK, 4), jnp.float32),
            pltpu.VMEM((CHUNK, 4), jnp.float32),
            pltpu.VMEM((CHUNK, 8), jnp.float32),
            pltpu.VMEM((16 * N_B,), jnp.float32),
            pltpu.VMEM((N_B,), jnp.float32),
            pltpu.SemaphoreType.DMA,
        ],
    )
    partials = grid_kernel(m[0], m[1], m[2], m[3], mb, pos4, atom_types, tab)
    return partials.reshape(NW, N_B).sum(axis=0)
